# Initial kernel scaffold; baseline (speedup 1.0000x reference)
import jax
import jax.numpy as jnp
from jax.experimental import pallas as pl


def kernel(node_feat, src, dst, last_emb1, last_emb2, W1, b1, W2, b2, gru1_Wi, gru1_Wh, gru1_bi, gru1_bh, gru2_Wi, gru2_Wh, gru2_bi, gru2_bh, num_current_edges, num_previous_edges):
    raise NotImplementedError("write your pallas kernel here")


# speedup vs baseline: 8.0149x; 8.0149x over previous
import functools

import jax
import jax.numpy as jnp
from jax import lax
from jax.experimental import pallas as pl
from jax.experimental.pallas import tpu as pltpu
from jax.experimental.pallas import tpu_sc as plsc

_N = 10000
_E = 320000
_H = 128
_NC = 2
_NS = 16
_CW = 128
_CPT = 80
_EP = _NC * _NS * _CPT * _CW
_NP = 10240
_RPT = _NP // _NS

@functools.cache
def _build_sc_kernels():
    mesh = plsc.VectorSubcoreMesh(core_axis_name="c", subcore_axis_name="s",
                                  num_cores=_NC, num_subcores=_NS)

    @functools.partial(
        pl.kernel,
        out_type=jax.ShapeDtypeStruct((_NC, _NP, _H), jnp.float32),
        mesh=mesh,
        scratch_types=[
            pltpu.VMEM((_CPT, _CW), jnp.int32),
            pltpu.VMEM((_CW, _H), jnp.float32),
            pltpu.VMEM_SHARED((_NP, _H), jnp.float32),
        ],
    )
    def _degree(dstr, zeros_nh, ones_h, degp, dst_idx, ones_v, acc):
        c = lax.axis_index("c")
        s = lax.axis_index("s")
        pltpu.sync_copy(zeros_nh.at[pl.ds(s * _RPT, _RPT)],
                        acc.at[pl.ds(s * _RPT, _RPT)])
        pltpu.sync_copy(ones_h, ones_v)
        base = (c * _NS + s) * _CPT
        pltpu.sync_copy(dstr.at[pl.ds(base, _CPT)], dst_idx)
        plsc.subcore_barrier()

        def _acc(j, carry):
            pltpu.sync_copy(ones_v, acc.at[dst_idx.at[j]], add=True)
            return carry

        lax.fori_loop(0, _CPT, _acc, 0)
        plsc.subcore_barrier()
        pltpu.sync_copy(acc.at[pl.ds(s * _RPT, _RPT)],
                        degp.at[c, pl.ds(s * _RPT, _RPT)])

    @functools.partial(
        pl.kernel,
        out_type=jax.ShapeDtypeStruct((_NC, _NP, _H), jnp.float32),
        mesh=mesh,
        scratch_types=[
            pltpu.VMEM((_CPT, _CW), jnp.int32),
            pltpu.VMEM((_CPT, _CW), jnp.int32),
            pltpu.VMEM((_CW, _H), jnp.float32),
            pltpu.VMEM_SHARED((_NP, _H), jnp.float32),
            pltpu.SemaphoreType.DMA,
        ],
    )
    def _aggregate(y, srcr, dstr, zeros_nh, aggp,
                   src_idx, dst_idx, rows, acc, sem):
        c = lax.axis_index("c")
        s = lax.axis_index("s")
        pltpu.sync_copy(zeros_nh.at[pl.ds(s * _RPT, _RPT)],
                        acc.at[pl.ds(s * _RPT, _RPT)])
        base = (c * _NS + s) * _CPT
        pltpu.sync_copy(srcr.at[pl.ds(base, _CPT)], src_idx)
        pltpu.sync_copy(dstr.at[pl.ds(base, _CPT)], dst_idx)
        plsc.subcore_barrier()

        def _edge(j, carry):
            pltpu.async_copy(y.at[src_idx.at[j]], rows, sem).wait()
            pltpu.sync_copy(rows, acc.at[dst_idx.at[j]], add=True)
            return carry

        lax.fori_loop(0, _CPT, _edge, 0)
        plsc.subcore_barrier()
        pltpu.sync_copy(acc.at[pl.ds(s * _RPT, _RPT)],
                        aggp.at[c, pl.ds(s * _RPT, _RPT)])

    return _degree, _aggregate


def _sc_degree(dstr, zeros_nh, ones_h):
    return _build_sc_kernels()[0](dstr, zeros_nh, ones_h)


def _sc_aggregate(y, srcr, dstr, zeros_nh):
    return _build_sc_kernels()[1](y, srcr, dstr, zeros_nh)


_R = 1000
_G = _N // _R


def _dinv_from_partials(degp_blk):
    deg = (jnp.sum(degp_blk[0], axis=1, keepdims=True)
           + jnp.sum(degp_blk[1], axis=1, keepdims=True)) * (1.0 / _H) + 1.0
    return lax.rsqrt(jnp.maximum(deg, 1.0))


def _tc_prescale_body(x_ref, w_ref, degp_ref, y_ref):
    dinv = _dinv_from_partials(degp_ref)
    y_ref[...] = jnp.dot(x_ref[...], w_ref[...],
                         preferred_element_type=jnp.float32) * dinv


def _gru_relu(conv, h, wi, wh, bi, bh):
    gi = jnp.dot(conv, wi, preferred_element_type=jnp.float32) + bi
    gh = jnp.dot(h, wh, preferred_element_type=jnp.float32) + bh
    r = jax.nn.sigmoid(gi[:, 0:_H] + gh[:, 0:_H])
    z = jax.nn.sigmoid(gi[:, _H:2 * _H] + gh[:, _H:2 * _H])
    n = jnp.tanh(gi[:, 2 * _H:3 * _H] + r * gh[:, 2 * _H:3 * _H])
    return jnp.maximum((1.0 - z) * n + z * h, 0.0)


def _tc_layer1_body(p_ref, y_ref, degp_ref, b_ref, h_ref, wi_ref, wh_ref,
                    bi_ref, bh_ref, w2_ref, e_ref, y2_ref):
    dinv = _dinv_from_partials(degp_ref)
    conv = (p_ref[0] + p_ref[1] + y_ref[...]) * dinv + b_ref[...]
    e = _gru_relu(conv, h_ref[...], wi_ref[...], wh_ref[...],
                  bi_ref[...], bh_ref[...])
    e_ref[...] = e
    y2_ref[...] = jnp.dot(e, w2_ref[...],
                          preferred_element_type=jnp.float32) * dinv


def _tc_layer2_body(p_ref, y_ref, degp_ref, b_ref, h_ref, wi_ref, wh_ref,
                    bi_ref, bh_ref, e_ref):
    dinv = _dinv_from_partials(degp_ref)
    conv = (p_ref[0] + p_ref[1] + y_ref[...]) * dinv + b_ref[...]
    e_ref[...] = _gru_relu(conv, h_ref[...], wi_ref[...], wh_ref[...],
                           bi_ref[...], bh_ref[...])


_row_spec = pl.BlockSpec((_R, _H), lambda i: (i, 0))
_p_spec = pl.BlockSpec((_NC, _R, _H), lambda i: (0, i, 0))
_degp_spec = pl.BlockSpec((_NC, _R, _H), lambda i: (0, i, 0))
_w_spec = pl.BlockSpec((_H, _H), lambda i: (0, 0))
_wg_spec = pl.BlockSpec((_H, 3 * _H), lambda i: (0, 0))
_b_spec = pl.BlockSpec((1, _H), lambda i: (0, 0))
_bg_spec = pl.BlockSpec((1, 3 * _H), lambda i: (0, 0))

_f32 = jnp.float32

_tc_prescale = pl.pallas_call(
    _tc_prescale_body,
    grid=(_G,),
    in_specs=[_row_spec, _w_spec, _degp_spec],
    out_specs=_row_spec,
    out_shape=jax.ShapeDtypeStruct((_N, _H), _f32),
)

_tc_layer1 = pl.pallas_call(
    _tc_layer1_body,
    grid=(_G,),
    in_specs=[_p_spec, _row_spec, _degp_spec, _b_spec, _row_spec,
              _wg_spec, _wg_spec, _bg_spec, _bg_spec, _w_spec],
    out_specs=[_row_spec, _row_spec],
    out_shape=[jax.ShapeDtypeStruct((_N, _H), _f32),
               jax.ShapeDtypeStruct((_N, _H), _f32)],
)

_tc_layer2 = pl.pallas_call(
    _tc_layer2_body,
    grid=(_G,),
    in_specs=[_p_spec, _row_spec, _degp_spec, _b_spec, _row_spec,
              _wg_spec, _wg_spec, _bg_spec, _bg_spec],
    out_specs=_row_spec,
    out_shape=jax.ShapeDtypeStruct((_N, _H), _f32),
)


def kernel(node_feat, src, dst, last_emb1, last_emb2, W1, b1, W2, b2,
           gru1_Wi, gru1_Wh, gru1_bi, gru1_bh,
           gru2_Wi, gru2_Wh, gru2_bi, gru2_bh,
           num_current_edges=_E, num_previous_edges=300000):
    pad = _EP - _E
    srcr = jnp.concatenate(
        [src.astype(jnp.int32), jnp.zeros((pad,), jnp.int32)]
    ).reshape(_EP // _CW, _CW)
    dstr = jnp.concatenate(
        [dst.astype(jnp.int32), jnp.full((pad,), _N, jnp.int32)]
    ).reshape(_EP // _CW, _CW)
    zeros_nh = jnp.zeros((_NP, _H), _f32)
    ones_h = jnp.ones((_CW, _H), _f32)

    degp = _sc_degree(dstr, zeros_nh, ones_h)
    y1 = _tc_prescale(node_feat, W1, degp)
    aggp1 = _sc_aggregate(y1, srcr, dstr, zeros_nh)
    e1, y2 = _tc_layer1(aggp1, y1, degp, b1.reshape(1, _H), last_emb1,
                        gru1_Wi, gru1_Wh, gru1_bi.reshape(1, 3 * _H),
                        gru1_bh.reshape(1, 3 * _H), W2)
    aggp2 = _sc_aggregate(y2, srcr, dstr, zeros_nh)
    e2 = _tc_layer2(aggp2, y2, degp, b2.reshape(1, _H), last_emb2,
                    gru2_Wi, gru2_Wh, gru2_bi.reshape(1, 3 * _H),
                    gru2_bh.reshape(1, 3 * _H))
    return (e1, e2)


# speedup vs baseline: 9.0923x; 1.1344x over previous
import functools

import jax
import jax.numpy as jnp
from jax import lax
from jax.experimental import pallas as pl
from jax.experimental.pallas import tpu as pltpu
from jax.experimental.pallas import tpu_sc as plsc

_N = 10000
_E = 320000
_H = 128
_NC = 2
_NS = 16
_CW = 64
_CPT = 160
_NPH = 2
_CPP = _CPT // _NPH
_EP = _NC * _NS * _CPT * _CW
_NP = 10240
_RPT = _NP // _NS

@functools.cache
def _build_sc_kernels():
    mesh = plsc.VectorSubcoreMesh(core_axis_name="c", subcore_axis_name="s",
                                  num_cores=_NC, num_subcores=_NS)

    _B = 2

    _sc_scratch = [
        pltpu.VMEM((_CPP, _CW), jnp.int32),
        pltpu.VMEM((_CPP, _CW), jnp.int32),
        pltpu.VMEM((_B, _CW, _H), jnp.float32),
        pltpu.VMEM_SHARED((_NP, _H), jnp.float32),
        [pltpu.SemaphoreType.DMA] * _B,
        [pltpu.SemaphoreType.DMA] * _B,
    ]

    @functools.partial(
        pl.kernel,
        out_type=jax.ShapeDtypeStruct((_NC, _NP, _H), jnp.float32),
        mesh=mesh,
        scratch_types=_sc_scratch,
    )
    def _degree(dstr, zeros_nh, ones_h, degp,
                src_idx, dst_idx, rows, acc, gsem, ssem):
        del src_idx, ssem
        c = lax.axis_index("c")
        s = lax.axis_index("s")
        pltpu.sync_copy(zeros_nh.at[pl.ds(s * _RPT, _RPT)],
                        acc.at[pl.ds(s * _RPT, _RPT)])
        pltpu.sync_copy(ones_h, rows.at[0])
        base = (c * _NS + s) * _CPT
        plsc.subcore_barrier()

        grp = 8

        def _acc(g, carry):
            for b in range(grp):
                j = g * grp + b
                pltpu.async_copy(rows.at[0], acc.at[dst_idx.at[j]], gsem[0],
                                 add=True)
            for b in range(grp):
                j = g * grp + b
                pltpu.make_async_copy(rows.at[0], acc.at[dst_idx.at[j]],
                                      gsem[0]).wait()
            return carry

        def _phase(h, carry):
            hb = pl.multiple_of(base + h * _CPP, 8)
            pltpu.sync_copy(dstr.at[pl.ds(hb, _CPP)], dst_idx)
            lax.fori_loop(0, _CPP // grp, _acc, 0)
            return carry

        lax.fori_loop(0, _NPH, _phase, 0)
        plsc.subcore_barrier()
        pltpu.sync_copy(acc.at[pl.ds(s * _RPT, _RPT)],
                        degp.at[c, pl.ds(s * _RPT, _RPT)])

    @functools.partial(
        pl.kernel,
        out_type=jax.ShapeDtypeStruct((_NC, _NP, _H), jnp.float32),
        mesh=mesh,
        scratch_types=_sc_scratch,
    )
    def _aggregate(y, srcr, dstr, zeros_nh, aggp,
                   src_idx, dst_idx, rows, acc, gsem, ssem):
        c = lax.axis_index("c")
        s = lax.axis_index("s")
        pltpu.sync_copy(zeros_nh.at[pl.ds(s * _RPT, _RPT)],
                        acc.at[pl.ds(s * _RPT, _RPT)])
        base = (c * _NS + s) * _CPT
        plsc.subcore_barrier()

        def _gather(j, b):
            pltpu.async_copy(y.at[src_idx.at[j]], rows.at[b], gsem[b])

        def _gather_wait(b):
            pltpu.make_async_copy(y.at[src_idx.at[0]], rows.at[b],
                                  gsem[b]).wait()

        def _scatter(j, b):
            pltpu.async_copy(rows.at[b], acc.at[dst_idx.at[j]], ssem[b],
                             add=True)

        def _scatter_wait(b):
            pltpu.make_async_copy(rows.at[b], acc.at[dst_idx.at[0]],
                                  ssem[b]).wait()

        def _group(g, carry):
            for b in range(_B):
                j = g * _B + b
                bp = (b - 1) % _B

                def _refill():
                    _scatter_wait(bp)

                    @pl.when(j + _B - 1 < _CPP)
                    def _():
                        _gather(j + _B - 1, bp)

                if b == 0:
                    pl.when(g > 0)(_refill)
                else:
                    _refill()
                _gather_wait(b)
                _scatter(j, b)
            return carry

        def _phase(h, carry):
            hb = pl.multiple_of(base + h * _CPP, 8)
            pltpu.sync_copy(srcr.at[pl.ds(hb, _CPP)], src_idx)
            pltpu.sync_copy(dstr.at[pl.ds(hb, _CPP)], dst_idx)
            for b in range(_B):
                _gather(b, b)
            lax.fori_loop(0, _CPP // _B, _group, 0)
            _scatter_wait((_CPP - 1) % _B)
            return carry

        lax.fori_loop(0, _NPH, _phase, 0)
        plsc.subcore_barrier()
        pltpu.sync_copy(acc.at[pl.ds(s * _RPT, _RPT)],
                        aggp.at[c, pl.ds(s * _RPT, _RPT)])

    return _degree, _aggregate


def _sc_degree(dstr, zeros_nh, ones_h):
    return _build_sc_kernels()[0](dstr, zeros_nh, ones_h)


def _sc_aggregate(y, srcr, dstr, zeros_nh):
    return _build_sc_kernels()[1](y, srcr, dstr, zeros_nh)


_R = 1000
_G = _N // _R


def _dinv_from_partials(degp_blk):
    deg = (jnp.sum(degp_blk[0], axis=1, keepdims=True)
           + jnp.sum(degp_blk[1], axis=1, keepdims=True)) * (1.0 / _H) + 1.0
    return lax.rsqrt(jnp.maximum(deg, 1.0))


def _tc_prescale_body(x_ref, w_ref, degp_ref, y_ref):
    dinv = _dinv_from_partials(degp_ref)
    y_ref[...] = jnp.dot(x_ref[...], w_ref[...],
                         preferred_element_type=jnp.float32) * dinv


def _gru_relu(conv, h, wi, wh, bi, bh):
    gi = jnp.dot(conv, wi, preferred_element_type=jnp.float32) + bi
    gh = jnp.dot(h, wh, preferred_element_type=jnp.float32) + bh
    r = jax.nn.sigmoid(gi[:, 0:_H] + gh[:, 0:_H])
    z = jax.nn.sigmoid(gi[:, _H:2 * _H] + gh[:, _H:2 * _H])
    n = jnp.tanh(gi[:, 2 * _H:3 * _H] + r * gh[:, 2 * _H:3 * _H])
    return jnp.maximum((1.0 - z) * n + z * h, 0.0)


def _tc_layer1_body(p_ref, y_ref, degp_ref, b_ref, h_ref, wi_ref, wh_ref,
                    bi_ref, bh_ref, w2_ref, e_ref, y2_ref):
    dinv = _dinv_from_partials(degp_ref)
    conv = (p_ref[0] + p_ref[1] + y_ref[...]) * dinv + b_ref[...]
    e = _gru_relu(conv, h_ref[...], wi_ref[...], wh_ref[...],
                  bi_ref[...], bh_ref[...])
    e_ref[...] = e
    y2_ref[...] = jnp.dot(e, w2_ref[...],
                          preferred_element_type=jnp.float32) * dinv


def _tc_layer2_body(p_ref, y_ref, degp_ref, b_ref, h_ref, wi_ref, wh_ref,
                    bi_ref, bh_ref, e_ref):
    dinv = _dinv_from_partials(degp_ref)
    conv = (p_ref[0] + p_ref[1] + y_ref[...]) * dinv + b_ref[...]
    e_ref[...] = _gru_relu(conv, h_ref[...], wi_ref[...], wh_ref[...],
                           bi_ref[...], bh_ref[...])


_row_spec = pl.BlockSpec((_R, _H), lambda i: (i, 0))
_p_spec = pl.BlockSpec((_NC, _R, _H), lambda i: (0, i, 0))
_degp_spec = pl.BlockSpec((_NC, _R, _H), lambda i: (0, i, 0))
_w_spec = pl.BlockSpec((_H, _H), lambda i: (0, 0))
_wg_spec = pl.BlockSpec((_H, 3 * _H), lambda i: (0, 0))
_b_spec = pl.BlockSpec((1, _H), lambda i: (0, 0))
_bg_spec = pl.BlockSpec((1, 3 * _H), lambda i: (0, 0))

_f32 = jnp.float32

_tc_prescale = pl.pallas_call(
    _tc_prescale_body,
    grid=(_G,),
    in_specs=[_row_spec, _w_spec, _degp_spec],
    out_specs=_row_spec,
    out_shape=jax.ShapeDtypeStruct((_N, _H), _f32),
)

_tc_layer1 = pl.pallas_call(
    _tc_layer1_body,
    grid=(_G,),
    in_specs=[_p_spec, _row_spec, _degp_spec, _b_spec, _row_spec,
              _wg_spec, _wg_spec, _bg_spec, _bg_spec, _w_spec],
    out_specs=[_row_spec, _row_spec],
    out_shape=[jax.ShapeDtypeStruct((_N, _H), _f32),
               jax.ShapeDtypeStruct((_N, _H), _f32)],
)

_tc_layer2 = pl.pallas_call(
    _tc_layer2_body,
    grid=(_G,),
    in_specs=[_p_spec, _row_spec, _degp_spec, _b_spec, _row_spec,
              _wg_spec, _wg_spec, _bg_spec, _bg_spec],
    out_specs=_row_spec,
    out_shape=jax.ShapeDtypeStruct((_N, _H), _f32),
)


def kernel(node_feat, src, dst, last_emb1, last_emb2, W1, b1, W2, b2,
           gru1_Wi, gru1_Wh, gru1_bi, gru1_bh,
           gru2_Wi, gru2_Wh, gru2_bi, gru2_bh,
           num_current_edges=_E, num_previous_edges=300000):
    pad = _EP - _E
    srcr = jnp.concatenate(
        [src.astype(jnp.int32), jnp.zeros((pad,), jnp.int32)]
    ).reshape(_EP // _CW, _CW)
    dstr = jnp.concatenate(
        [dst.astype(jnp.int32), jnp.full((pad,), _N, jnp.int32)]
    ).reshape(_EP // _CW, _CW)
    zeros_nh = jnp.zeros((_NP, _H), _f32)
    ones_h = jnp.ones((_CW, _H), _f32)

    degp = _sc_degree(dstr, zeros_nh, ones_h)
    y1 = _tc_prescale(node_feat, W1, degp)
    aggp1 = _sc_aggregate(y1, srcr, dstr, zeros_nh)
    e1, y2 = _tc_layer1(aggp1, y1, degp, b1.reshape(1, _H), last_emb1,
                        gru1_Wi, gru1_Wh, gru1_bi.reshape(1, 3 * _H),
                        gru1_bh.reshape(1, 3 * _H), W2)
    aggp2 = _sc_aggregate(y2, srcr, dstr, zeros_nh)
    e2 = _tc_layer2(aggp2, y2, degp, b2.reshape(1, _H), last_emb2,
                    gru2_Wi, gru2_Wh, gru2_bi.reshape(1, 3 * _H),
                    gru2_bh.reshape(1, 3 * _H))
    return (e1, e2)


# speedup vs baseline: 10.3505x; 1.1384x over previous
import functools

import jax
import jax.numpy as jnp
from jax import lax
from jax.experimental import pallas as pl
from jax.experimental.pallas import tpu as pltpu
from jax.experimental.pallas import tpu_sc as plsc

_N = 10000
_E = 320000
_H = 128
_NC = 2
_NS = 16
_CW = 64
_CPT = 160
_NPH = 2
_CPP = _CPT // _NPH
_EP = _NC * _NS * _CPT * _CW
_NP = 10240
_RPT = _NP // _NS

@functools.cache
def _build_sc_kernels():
    mesh = plsc.VectorSubcoreMesh(core_axis_name="c", subcore_axis_name="s",
                                  num_cores=_NC, num_subcores=_NS)

    _B = 2

    _sc_scratch = [
        pltpu.VMEM((_CPP, _CW), jnp.int32),
        pltpu.VMEM((_CPP, _CW), jnp.int32),
        pltpu.VMEM((_B, _CW, _H), jnp.float32),
        pltpu.VMEM_SHARED((_NP, _H), jnp.float32),
        [pltpu.SemaphoreType.DMA] * _B,
        [pltpu.SemaphoreType.DMA] * _B,
    ]

    @functools.partial(
        pl.kernel,
        out_type=jax.ShapeDtypeStruct((_NC, _NP, _H), jnp.float32),
        mesh=mesh,
        scratch_types=_sc_scratch,
    )
    def _degree(dstr, zeros_nh, ones_h, degp,
                src_idx, dst_idx, rows, acc, gsem, ssem):
        del src_idx, ssem
        c = lax.axis_index("c")
        s = lax.axis_index("s")
        pltpu.sync_copy(zeros_nh.at[pl.ds(s * _RPT, _RPT)],
                        acc.at[pl.ds(s * _RPT, _RPT)])
        pltpu.sync_copy(ones_h, rows.at[0])
        base = (c * _NS + s) * _CPT
        plsc.subcore_barrier()

        grp = 8

        def _acc(g, carry):
            for b in range(grp):
                j = g * grp + b
                pltpu.async_copy(rows.at[0], acc.at[dst_idx.at[j]], gsem[0],
                                 add=True)
            for b in range(grp):
                j = g * grp + b
                pltpu.make_async_copy(rows.at[0], acc.at[dst_idx.at[j]],
                                      gsem[0]).wait()
            return carry

        def _phase(h, carry):
            hb = pl.multiple_of(base + h * _CPP, 8)
            pltpu.sync_copy(dstr.at[pl.ds(hb, _CPP)], dst_idx)
            lax.fori_loop(0, _CPP // grp, _acc, 0)
            return carry

        lax.fori_loop(0, _NPH, _phase, 0)
        plsc.subcore_barrier()
        pltpu.sync_copy(acc.at[pl.ds(s * _RPT, _RPT)],
                        degp.at[c, pl.ds(s * _RPT, _RPT)])

    @functools.partial(
        pl.kernel,
        out_type=jax.ShapeDtypeStruct((_NC, _NP, _H), jnp.float32),
        mesh=mesh,
        scratch_types=_sc_scratch,
    )
    def _aggregate(y, srcr, dstr, zeros_nh, aggp,
                   src_idx, dst_idx, rows, acc, gsem, ssem):
        c = lax.axis_index("c")
        s = lax.axis_index("s")
        pltpu.sync_copy(zeros_nh.at[pl.ds(s * _RPT, _RPT)],
                        acc.at[pl.ds(s * _RPT, _RPT)])
        base = (c * _NS + s) * _CPT
        plsc.subcore_barrier()

        def _gather(j, b):
            pltpu.async_copy(y.at[src_idx.at[j]], rows.at[b], gsem[b])

        def _gather_wait(b):
            pltpu.make_async_copy(y.at[src_idx.at[0]], rows.at[b],
                                  gsem[b]).wait()

        def _scatter(j, b):
            pltpu.async_copy(rows.at[b], acc.at[dst_idx.at[j]], ssem[b],
                             add=True)

        def _scatter_wait(b):
            pltpu.make_async_copy(rows.at[b], acc.at[dst_idx.at[0]],
                                  ssem[b]).wait()

        def _group(g, carry):
            for b in range(_B):
                j = g * _B + b
                bp = (b - 1) % _B

                def _refill():
                    _scatter_wait(bp)

                    @pl.when(j + _B - 1 < _CPP)
                    def _():
                        _gather(j + _B - 1, bp)

                if b == 0:
                    pl.when(g > 0)(_refill)
                else:
                    _refill()
                _gather_wait(b)
                _scatter(j, b)
            return carry

        def _phase(h, carry):
            hb = pl.multiple_of(base + h * _CPP, 8)
            pltpu.sync_copy(srcr.at[pl.ds(hb, _CPP)], src_idx)
            pltpu.sync_copy(dstr.at[pl.ds(hb, _CPP)], dst_idx)
            for b in range(_B):
                _gather(b, b)
            lax.fori_loop(0, _CPP // _B, _group, 0)
            _scatter_wait((_CPP - 1) % _B)
            return carry

        lax.fori_loop(0, _NPH, _phase, 0)
        plsc.subcore_barrier()
        pltpu.sync_copy(acc.at[pl.ds(s * _RPT, _RPT)],
                        aggp.at[c, pl.ds(s * _RPT, _RPT)])

    return _degree, _aggregate


def _sc_degree(dstr, zeros_nh, ones_h):
    return _build_sc_kernels()[0](dstr, zeros_nh, ones_h)


def _sc_aggregate(y, srcr, dstr, zeros_nh):
    return _build_sc_kernels()[1](y, srcr, dstr, zeros_nh)


_R = 1000
_G = _N // _R


def _dinv_from_partials(degp_blk):
    deg = (jnp.sum(degp_blk[0], axis=1, keepdims=True)
           + jnp.sum(degp_blk[1], axis=1, keepdims=True)) * (1.0 / _H) + 1.0
    return lax.rsqrt(jnp.maximum(deg, 1.0))


def _tc_prescale_body(x_ref, w_ref, degp_ref, y_ref):
    dinv = _dinv_from_partials(degp_ref)
    y_ref[...] = jnp.dot(x_ref[...], w_ref[...],
                         preferred_element_type=jnp.float32) * dinv


def _gru_relu(conv, h, wi, wh, bi, bh):
    gi = jnp.dot(conv, wi, preferred_element_type=jnp.float32) + bi
    gh = jnp.dot(h, wh, preferred_element_type=jnp.float32) + bh
    r = jax.nn.sigmoid(gi[:, 0:_H] + gh[:, 0:_H])
    z = jax.nn.sigmoid(gi[:, _H:2 * _H] + gh[:, _H:2 * _H])
    n = jnp.tanh(gi[:, 2 * _H:3 * _H] + r * gh[:, 2 * _H:3 * _H])
    return jnp.maximum((1.0 - z) * n + z * h, 0.0)


def _tc_layer1_body(p_ref, y_ref, degp_ref, b_ref, h_ref, wi_ref, wh_ref,
                    bi_ref, bh_ref, w2_ref, e_ref, y2_ref):
    dinv = _dinv_from_partials(degp_ref)
    conv = (p_ref[0] + p_ref[1] + y_ref[...]) * dinv + b_ref[...]
    e = _gru_relu(conv, h_ref[...], wi_ref[...], wh_ref[...],
                  bi_ref[...], bh_ref[...])
    e_ref[...] = e
    y2_ref[...] = jnp.dot(e, w2_ref[...],
                          preferred_element_type=jnp.float32) * dinv


def _tc_layer2_body(p_ref, y_ref, degp_ref, b_ref, h_ref, wi_ref, wh_ref,
                    bi_ref, bh_ref, e_ref):
    dinv = _dinv_from_partials(degp_ref)
    conv = (p_ref[0] + p_ref[1] + y_ref[...]) * dinv + b_ref[...]
    e_ref[...] = _gru_relu(conv, h_ref[...], wi_ref[...], wh_ref[...],
                           bi_ref[...], bh_ref[...])


_row_spec = pl.BlockSpec((_R, _H), lambda i: (i, 0))
_p_spec = pl.BlockSpec((_NC, _R, _H), lambda i: (0, i, 0))
_degp_spec = pl.BlockSpec((_NC, _R, _H), lambda i: (0, i, 0))
_w_spec = pl.BlockSpec((_H, _H), lambda i: (0, 0))
_wg_spec = pl.BlockSpec((_H, 3 * _H), lambda i: (0, 0))
_b_spec = pl.BlockSpec((1, _H), lambda i: (0, 0))
_bg_spec = pl.BlockSpec((1, 3 * _H), lambda i: (0, 0))

_f32 = jnp.float32

_tc_prescale = pl.pallas_call(
    _tc_prescale_body,
    grid=(_G,),
    in_specs=[_row_spec, _w_spec, _degp_spec],
    out_specs=_row_spec,
    out_shape=jax.ShapeDtypeStruct((_N, _H), _f32),
)

_tc_layer1 = pl.pallas_call(
    _tc_layer1_body,
    grid=(_G,),
    in_specs=[_p_spec, _row_spec, _degp_spec, _b_spec, _row_spec,
              _wg_spec, _wg_spec, _bg_spec, _bg_spec, _w_spec],
    out_specs=[_row_spec, _row_spec],
    out_shape=[jax.ShapeDtypeStruct((_N, _H), _f32),
               jax.ShapeDtypeStruct((_N, _H), _f32)],
)

_tc_layer2 = pl.pallas_call(
    _tc_layer2_body,
    grid=(_G,),
    in_specs=[_p_spec, _row_spec, _degp_spec, _b_spec, _row_spec,
              _wg_spec, _wg_spec, _bg_spec, _bg_spec],
    out_specs=_row_spec,
    out_shape=jax.ShapeDtypeStruct((_N, _H), _f32),
)


def kernel(node_feat, src, dst, last_emb1, last_emb2, W1, b1, W2, b2,
           gru1_Wi, gru1_Wh, gru1_bi, gru1_bh,
           gru2_Wi, gru2_Wh, gru2_bi, gru2_bh,
           num_current_edges=_E, num_previous_edges=300000):
    pad = _EP - _E
    nrow = _EP // _CW
    perm = jnp.concatenate([jnp.arange(0, nrow, 2, dtype=jnp.int32),
                            jnp.arange(1, nrow, 2, dtype=jnp.int32)])
    srcr = jnp.concatenate(
        [src.astype(jnp.int32), jnp.zeros((pad,), jnp.int32)]
    ).reshape(nrow, _CW)[perm]
    dstr = jnp.concatenate(
        [dst.astype(jnp.int32), jnp.full((pad,), _N, jnp.int32)]
    ).reshape(nrow, _CW)[perm]
    zeros_nh = jnp.zeros((_NP, _H), _f32)
    ones_h = jnp.ones((_CW, _H), _f32)

    degp = _sc_degree(dstr, zeros_nh, ones_h)
    y1 = _tc_prescale(node_feat, W1, degp)
    aggp1 = _sc_aggregate(y1, srcr, dstr, zeros_nh)
    e1, y2 = _tc_layer1(aggp1, y1, degp, b1.reshape(1, _H), last_emb1,
                        gru1_Wi, gru1_Wh, gru1_bi.reshape(1, 3 * _H),
                        gru1_bh.reshape(1, 3 * _H), W2)
    aggp2 = _sc_aggregate(y2, srcr, dstr, zeros_nh)
    e2 = _tc_layer2(aggp2, y2, degp, b2.reshape(1, _H), last_emb2,
                    gru2_Wi, gru2_Wh, gru2_bi.reshape(1, 3 * _H),
                    gru2_bh.reshape(1, 3 * _H))
    return (e1, e2)


# speedup vs baseline: 23.6889x; 2.2887x over previous
import functools

import jax
import jax.numpy as jnp
from jax import lax
from jax.experimental import pallas as pl
from jax.experimental.pallas import tpu as pltpu
from jax.experimental.pallas import tpu_sc as plsc

_N = 10000
_E = 320000
_H = 128
_NC = 2
_NS = 16
_CW = 64
_CPT = 160
_NPH = 2
_CPP = _CPT // _NPH
_EP = _NC * _NS * _CPT * _CW
_NP = 10240
_RPT = _NP // _NS

@functools.cache
def _build_sc_kernels():
    mesh = plsc.VectorSubcoreMesh(core_axis_name="c", subcore_axis_name="s",
                                  num_cores=_NC, num_subcores=_NS)

    _B = 2

    _sc_scratch = [
        pltpu.VMEM((_CPP, _CW), jnp.int32),
        pltpu.VMEM((_CPP, _CW), jnp.int32),
        pltpu.VMEM((_B, _CW, _H), jnp.float32),
        pltpu.VMEM_SHARED((_NP, _H), jnp.float32),
        [pltpu.SemaphoreType.DMA] * _B,
        [pltpu.SemaphoreType.DMA] * _B,
    ]

    @functools.partial(
        pl.kernel,
        out_type=jax.ShapeDtypeStruct((_NC, _NP, _H), jnp.float32),
        mesh=mesh,
        scratch_types=_sc_scratch,
    )
    def _degree(dstr, zeros_nh, ones_h, degp,
                src_idx, dst_idx, rows, acc, gsem, ssem):
        del src_idx, ssem
        c = lax.axis_index("c")
        s = lax.axis_index("s")
        pltpu.sync_copy(zeros_nh.at[pl.ds(s * _RPT, _RPT)],
                        acc.at[pl.ds(s * _RPT, _RPT)])
        pltpu.sync_copy(ones_h, rows.at[0])
        base = (c * _NS + s) * _CPT
        plsc.subcore_barrier()

        grp = 8

        def _acc(g, carry):
            for b in range(grp):
                j = g * grp + b
                pltpu.async_copy(rows.at[0], acc.at[dst_idx.at[j]], gsem[0],
                                 add=True)
            for b in range(grp):
                j = g * grp + b
                pltpu.make_async_copy(rows.at[0], acc.at[dst_idx.at[j]],
                                      gsem[0]).wait()
            return carry

        def _phase(h, carry):
            hb = pl.multiple_of(base + h * _CPP, 8)
            pltpu.sync_copy(dstr.at[pl.ds(hb, _CPP)], dst_idx)
            lax.fori_loop(0, _CPP // grp, _acc, 0)
            return carry

        lax.fori_loop(0, _NPH, _phase, 0)
        plsc.subcore_barrier()
        pltpu.sync_copy(acc.at[pl.ds(s * _RPT, _RPT)],
                        degp.at[c, pl.ds(s * _RPT, _RPT)])

    @functools.partial(
        pl.kernel,
        out_type=jax.ShapeDtypeStruct((_NC, _NP, _H), jnp.float32),
        mesh=mesh,
        scratch_types=_sc_scratch,
    )
    def _aggregate(y, srcr, dstr, zeros_nh, aggp,
                   src_idx, dst_idx, rows, acc, gsem, ssem):
        c = lax.axis_index("c")
        s = lax.axis_index("s")
        pltpu.sync_copy(zeros_nh.at[pl.ds(s * _RPT, _RPT)],
                        acc.at[pl.ds(s * _RPT, _RPT)])
        base = (c * _NS + s) * _CPT
        plsc.subcore_barrier()

        def _gather(j, b):
            pltpu.async_copy(y.at[src_idx.at[j]], rows.at[b], gsem[b])

        def _gather_wait(b):
            pltpu.make_async_copy(y.at[src_idx.at[0]], rows.at[b],
                                  gsem[b]).wait()

        def _scatter(j, b):
            pltpu.async_copy(rows.at[b], acc.at[dst_idx.at[j]], ssem[b],
                             add=True)

        def _scatter_wait(b):
            pltpu.make_async_copy(rows.at[b], acc.at[dst_idx.at[0]],
                                  ssem[b]).wait()

        def _group(g, carry):
            for b in range(_B):
                j = g * _B + b
                bp = (b - 1) % _B

                def _refill():
                    _scatter_wait(bp)

                    @pl.when(j + _B - 1 < _CPP)
                    def _():
                        _gather(j + _B - 1, bp)

                if b == 0:
                    pl.when(g > 0)(_refill)
                else:
                    _refill()
                _gather_wait(b)
                _scatter(j, b)
            return carry

        def _phase(h, carry):
            hb = pl.multiple_of(base + h * _CPP, 8)
            pltpu.sync_copy(srcr.at[pl.ds(hb, _CPP)], src_idx)
            pltpu.sync_copy(dstr.at[pl.ds(hb, _CPP)], dst_idx)
            for b in range(_B):
                _gather(b, b)
            lax.fori_loop(0, _CPP // _B, _group, 0)
            _scatter_wait((_CPP - 1) % _B)
            return carry

        lax.fori_loop(0, _NPH, _phase, 0)
        plsc.subcore_barrier()
        pltpu.sync_copy(acc.at[pl.ds(s * _RPT, _RPT)],
                        aggp.at[c, pl.ds(s * _RPT, _RPT)])

    return _degree, _aggregate


def _sc_degree(dstr, zeros_nh, ones_h):
    return _build_sc_kernels()[0](dstr, zeros_nh, ones_h)


def _sc_aggregate(y, srcr, dstr, zeros_nh):
    return _build_sc_kernels()[1](y, srcr, dstr, zeros_nh)


_R = 1000
_G = _N // _R


def _dinv_from_partials(degp_blk):
    deg = (jnp.sum(degp_blk[0], axis=1, keepdims=True)
           + jnp.sum(degp_blk[1], axis=1, keepdims=True)) * (1.0 / _H) + 1.0
    return lax.rsqrt(jnp.maximum(deg, 1.0))


def _tc_prescale_body(x_ref, w_ref, degp_ref, y_ref):
    dinv = _dinv_from_partials(degp_ref)
    y_ref[...] = jnp.dot(x_ref[...], w_ref[...],
                         preferred_element_type=jnp.float32) * dinv


def _gru_relu(conv, h, wi, wh, bi, bh):
    gi = jnp.dot(conv, wi, preferred_element_type=jnp.float32) + bi
    gh = jnp.dot(h, wh, preferred_element_type=jnp.float32) + bh
    r = jax.nn.sigmoid(gi[:, 0:_H] + gh[:, 0:_H])
    z = jax.nn.sigmoid(gi[:, _H:2 * _H] + gh[:, _H:2 * _H])
    n = jnp.tanh(gi[:, 2 * _H:3 * _H] + r * gh[:, 2 * _H:3 * _H])
    return jnp.maximum((1.0 - z) * n + z * h, 0.0)


def _tc_layer1_body(p_ref, y_ref, degp_ref, b_ref, h_ref, wi_ref, wh_ref,
                    bi_ref, bh_ref, w2_ref, e_ref, y2_ref):
    dinv = _dinv_from_partials(degp_ref)
    conv = (p_ref[0] + p_ref[1] + y_ref[...]) * dinv + b_ref[...]
    e = _gru_relu(conv, h_ref[...], wi_ref[...], wh_ref[...],
                  bi_ref[...], bh_ref[...])
    e_ref[...] = e
    y2_ref[...] = jnp.dot(e, w2_ref[...],
                          preferred_element_type=jnp.float32) * dinv


def _tc_layer2_body(p_ref, y_ref, degp_ref, b_ref, h_ref, wi_ref, wh_ref,
                    bi_ref, bh_ref, e_ref):
    dinv = _dinv_from_partials(degp_ref)
    conv = (p_ref[0] + p_ref[1] + y_ref[...]) * dinv + b_ref[...]
    e_ref[...] = _gru_relu(conv, h_ref[...], wi_ref[...], wh_ref[...],
                           bi_ref[...], bh_ref[...])


_row_spec = pl.BlockSpec((_R, _H), lambda i: (i, 0))
_p_spec = pl.BlockSpec((_NC, _R, _H), lambda i: (0, i, 0))
_degp_spec = pl.BlockSpec((_NC, _R, _H), lambda i: (0, i, 0))
_w_spec = pl.BlockSpec((_H, _H), lambda i: (0, 0))
_wg_spec = pl.BlockSpec((_H, 3 * _H), lambda i: (0, 0))
_b_spec = pl.BlockSpec((1, _H), lambda i: (0, 0))
_bg_spec = pl.BlockSpec((1, 3 * _H), lambda i: (0, 0))

_f32 = jnp.float32

_tc_prescale = pl.pallas_call(
    _tc_prescale_body,
    grid=(_G,),
    in_specs=[_row_spec, _w_spec, _degp_spec],
    out_specs=_row_spec,
    out_shape=jax.ShapeDtypeStruct((_N, _H), _f32),
)

_tc_layer1 = pl.pallas_call(
    _tc_layer1_body,
    grid=(_G,),
    in_specs=[_p_spec, _row_spec, _degp_spec, _b_spec, _row_spec,
              _wg_spec, _wg_spec, _bg_spec, _bg_spec, _w_spec],
    out_specs=[_row_spec, _row_spec],
    out_shape=[jax.ShapeDtypeStruct((_N, _H), _f32),
               jax.ShapeDtypeStruct((_N, _H), _f32)],
)

_tc_layer2 = pl.pallas_call(
    _tc_layer2_body,
    grid=(_G,),
    in_specs=[_p_spec, _row_spec, _degp_spec, _b_spec, _row_spec,
              _wg_spec, _wg_spec, _bg_spec, _bg_spec],
    out_specs=_row_spec,
    out_shape=jax.ShapeDtypeStruct((_N, _H), _f32),
)


def kernel(node_feat, src, dst, last_emb1, last_emb2, W1, b1, W2, b2,
           gru1_Wi, gru1_Wh, gru1_bi, gru1_bh,
           gru2_Wi, gru2_Wh, gru2_bi, gru2_bh,
           num_current_edges=_E, num_previous_edges=300000):
    pad = _EP - _E
    nrow = _EP // _CW
    perm = jnp.concatenate([jnp.arange(0, nrow, 2, dtype=jnp.int32),
                            jnp.arange(1, nrow, 2, dtype=jnp.int32)])
    pad_src = jnp.arange(pad, dtype=jnp.int32) % _N
    pad_dst = _N + (jnp.arange(pad, dtype=jnp.int32) % (_NP - _N))
    srcr = jnp.concatenate(
        [src.astype(jnp.int32), pad_src]
    ).reshape(nrow, _CW)[perm]
    dstr = jnp.concatenate(
        [dst.astype(jnp.int32), pad_dst]
    ).reshape(nrow, _CW)[perm]
    zeros_nh = jnp.zeros((_NP, _H), _f32)
    ones_h = jnp.ones((_CW, _H), _f32)

    degp = _sc_degree(dstr, zeros_nh, ones_h)
    y1 = _tc_prescale(node_feat, W1, degp)
    aggp1 = _sc_aggregate(y1, srcr, dstr, zeros_nh)
    e1, y2 = _tc_layer1(aggp1, y1, degp, b1.reshape(1, _H), last_emb1,
                        gru1_Wi, gru1_Wh, gru1_bi.reshape(1, 3 * _H),
                        gru1_bh.reshape(1, 3 * _H), W2)
    aggp2 = _sc_aggregate(y2, srcr, dstr, zeros_nh)
    e2 = _tc_layer2(aggp2, y2, degp, b2.reshape(1, _H), last_emb2,
                    gru2_Wi, gru2_Wh, gru2_bi.reshape(1, 3 * _H),
                    gru2_bh.reshape(1, 3 * _H))
    return (e1, e2)


# speedup vs baseline: 24.1270x; 1.0185x over previous
import functools

import jax
import jax.numpy as jnp
from jax import lax
from jax.experimental import pallas as pl
from jax.experimental.pallas import tpu as pltpu
from jax.experimental.pallas import tpu_sc as plsc

_N = 10000
_E = 320000
_H = 128
_NC = 2
_NS = 16
_CW = 64
_CPT = 160
_NPH = 2
_CPP = _CPT // _NPH
_EP = _NC * _NS * _CPT * _CW
_NP = 10240
_RPT = _NP // _NS

@functools.cache
def _build_sc_kernels():
    mesh = plsc.VectorSubcoreMesh(core_axis_name="c", subcore_axis_name="s",
                                  num_cores=_NC, num_subcores=_NS)

    _B = 2

    _sc_scratch = [
        pltpu.VMEM((_CPP, _CW), jnp.int32),
        pltpu.VMEM((_CPP, _CW), jnp.int32),
        pltpu.VMEM((_B, _CW, _H), jnp.float32),
        pltpu.VMEM_SHARED((_NP, _H), jnp.float32),
        [pltpu.SemaphoreType.DMA] * _B,
        [pltpu.SemaphoreType.DMA] * _B,
    ]

    @functools.partial(
        pl.kernel,
        out_type=jax.ShapeDtypeStruct((_NC, _NP, _H), jnp.float32),
        mesh=mesh,
        scratch_types=_sc_scratch,
    )
    def _degree(dstr, zeros_nh, ones_h, degp,
                src_idx, dst_idx, rows, acc, gsem, ssem):
        del src_idx, ssem
        c = lax.axis_index("c")
        s = lax.axis_index("s")
        pltpu.sync_copy(zeros_nh.at[pl.ds(s * _RPT, _RPT)],
                        acc.at[pl.ds(s * _RPT, _RPT)])
        pltpu.sync_copy(ones_h, rows.at[0])
        plsc.subcore_barrier()

        grp = 8

        def _acc(g, carry):
            for b in range(grp):
                j = g * grp + b
                pltpu.async_copy(rows.at[0], acc.at[dst_idx.at[j]], gsem[0],
                                 add=True)
            for b in range(grp):
                j = g * grp + b
                pltpu.make_async_copy(rows.at[0], acc.at[dst_idx.at[j]],
                                      gsem[0]).wait()
            return carry

        def _phase(h, carry):
            hb = pl.multiple_of(((s * _NPH + h) * _NC + c) * _CPP, 8)
            pltpu.sync_copy(dstr.at[pl.ds(hb, _CPP)], dst_idx)
            lax.fori_loop(0, _CPP // grp, _acc, 0)
            return carry

        lax.fori_loop(0, _NPH, _phase, 0)
        plsc.subcore_barrier()
        pltpu.sync_copy(acc.at[pl.ds(s * _RPT, _RPT)],
                        degp.at[c, pl.ds(s * _RPT, _RPT)])

    @functools.partial(
        pl.kernel,
        out_type=jax.ShapeDtypeStruct((_NC, _NP, _H), jnp.float32),
        mesh=mesh,
        scratch_types=_sc_scratch,
    )
    def _aggregate(y, srcr, dstr, zeros_nh, aggp,
                   src_idx, dst_idx, rows, acc, gsem, ssem):
        c = lax.axis_index("c")
        s = lax.axis_index("s")
        pltpu.sync_copy(zeros_nh.at[pl.ds(s * _RPT, _RPT)],
                        acc.at[pl.ds(s * _RPT, _RPT)])
        plsc.subcore_barrier()

        def _gather(j, b):
            pltpu.async_copy(y.at[src_idx.at[j]], rows.at[b], gsem[b])

        def _gather_wait(b):
            pltpu.make_async_copy(y.at[src_idx.at[0]], rows.at[b],
                                  gsem[b]).wait()

        def _scatter(j, b):
            pltpu.async_copy(rows.at[b], acc.at[dst_idx.at[j]], ssem[b],
                             add=True)

        def _scatter_wait(b):
            pltpu.make_async_copy(rows.at[b], acc.at[dst_idx.at[0]],
                                  ssem[b]).wait()

        def _group(g, carry):
            for b in range(_B):
                j = g * _B + b
                bp = (b - 1) % _B

                def _refill():
                    _scatter_wait(bp)

                    @pl.when(j + _B - 1 < _CPP)
                    def _():
                        _gather(j + _B - 1, bp)

                if b == 0:
                    pl.when(g > 0)(_refill)
                else:
                    _refill()
                _gather_wait(b)
                _scatter(j, b)
            return carry

        def _phase(h, carry):
            hb = pl.multiple_of(((s * _NPH + h) * _NC + c) * _CPP, 8)
            pltpu.sync_copy(srcr.at[pl.ds(hb, _CPP)], src_idx)
            pltpu.sync_copy(dstr.at[pl.ds(hb, _CPP)], dst_idx)
            for b in range(_B):
                _gather(b, b)
            lax.fori_loop(0, _CPP // _B, _group, 0)
            _scatter_wait((_CPP - 1) % _B)
            return carry

        lax.fori_loop(0, _NPH, _phase, 0)
        plsc.subcore_barrier()
        pltpu.sync_copy(acc.at[pl.ds(s * _RPT, _RPT)],
                        aggp.at[c, pl.ds(s * _RPT, _RPT)])

    return _degree, _aggregate


def _sc_degree(dstr, zeros_nh, ones_h):
    return _build_sc_kernels()[0](dstr, zeros_nh, ones_h)


def _sc_aggregate(y, srcr, dstr, zeros_nh):
    return _build_sc_kernels()[1](y, srcr, dstr, zeros_nh)


_R = 1000
_G = _N // _R


def _dinv_from_partials(degp_blk):
    deg = (jnp.sum(degp_blk[0], axis=1, keepdims=True)
           + jnp.sum(degp_blk[1], axis=1, keepdims=True)) * (1.0 / _H) + 1.0
    return lax.rsqrt(jnp.maximum(deg, 1.0))


def _tc_prescale_body(x_ref, w_ref, degp_ref, y_ref, d8_ref):
    dinv = _dinv_from_partials(degp_ref)
    y_ref[...] = jnp.dot(x_ref[...], w_ref[...],
                         preferred_element_type=jnp.float32) * dinv
    d8_ref[...] = jnp.broadcast_to(dinv, (_R, 8))


def _dinv_from_d8(d8_ref):
    return jnp.sum(d8_ref[...], axis=1, keepdims=True) * 0.125


def _gru_relu(conv, h, wi, wh, bi, bh):
    gi = jnp.dot(conv, wi, preferred_element_type=jnp.float32) + bi
    gh = jnp.dot(h, wh, preferred_element_type=jnp.float32) + bh
    r = jax.nn.sigmoid(gi[:, 0:_H] + gh[:, 0:_H])
    z = jax.nn.sigmoid(gi[:, _H:2 * _H] + gh[:, _H:2 * _H])
    n = jnp.tanh(gi[:, 2 * _H:3 * _H] + r * gh[:, 2 * _H:3 * _H])
    return jnp.maximum((1.0 - z) * n + z * h, 0.0)


def _tc_layer1_body(p_ref, y_ref, d8_ref, b_ref, h_ref, wi_ref, wh_ref,
                    bi_ref, bh_ref, w2_ref, e_ref, y2_ref):
    dinv = _dinv_from_d8(d8_ref)
    conv = (p_ref[0] + p_ref[1] + y_ref[...]) * dinv + b_ref[...]
    e = _gru_relu(conv, h_ref[...], wi_ref[...], wh_ref[...],
                  bi_ref[...], bh_ref[...])
    e_ref[...] = e
    y2_ref[...] = jnp.dot(e, w2_ref[...],
                          preferred_element_type=jnp.float32) * dinv


def _tc_layer2_body(p_ref, y_ref, d8_ref, b_ref, h_ref, wi_ref, wh_ref,
                    bi_ref, bh_ref, e_ref):
    dinv = _dinv_from_d8(d8_ref)
    conv = (p_ref[0] + p_ref[1] + y_ref[...]) * dinv + b_ref[...]
    e_ref[...] = _gru_relu(conv, h_ref[...], wi_ref[...], wh_ref[...],
                           bi_ref[...], bh_ref[...])


_row_spec = pl.BlockSpec((_R, _H), lambda i: (i, 0))
_p_spec = pl.BlockSpec((_NC, _R, _H), lambda i: (0, i, 0))
_degp_spec = pl.BlockSpec((_NC, _R, _H), lambda i: (0, i, 0))
_w_spec = pl.BlockSpec((_H, _H), lambda i: (0, 0))
_wg_spec = pl.BlockSpec((_H, 3 * _H), lambda i: (0, 0))
_b_spec = pl.BlockSpec((1, _H), lambda i: (0, 0))
_d8_spec = pl.BlockSpec((_R, 8), lambda i: (i, 0))
_bg_spec = pl.BlockSpec((1, 3 * _H), lambda i: (0, 0))

_f32 = jnp.float32

_tc_prescale = pl.pallas_call(
    _tc_prescale_body,
    grid=(_G,),
    in_specs=[_row_spec, _w_spec, _degp_spec],
    out_specs=[_row_spec, _d8_spec],
    out_shape=[jax.ShapeDtypeStruct((_N, _H), _f32),
               jax.ShapeDtypeStruct((_N, 8), _f32)],
)

_tc_layer1 = pl.pallas_call(
    _tc_layer1_body,
    grid=(_G,),
    in_specs=[_p_spec, _row_spec, _d8_spec, _b_spec, _row_spec,
              _wg_spec, _wg_spec, _bg_spec, _bg_spec, _w_spec],
    out_specs=[_row_spec, _row_spec],
    out_shape=[jax.ShapeDtypeStruct((_N, _H), _f32),
               jax.ShapeDtypeStruct((_N, _H), _f32)],
)

_tc_layer2 = pl.pallas_call(
    _tc_layer2_body,
    grid=(_G,),
    in_specs=[_p_spec, _row_spec, _d8_spec, _b_spec, _row_spec,
              _wg_spec, _wg_spec, _bg_spec, _bg_spec],
    out_specs=_row_spec,
    out_shape=jax.ShapeDtypeStruct((_N, _H), _f32),
)


def kernel(node_feat, src, dst, last_emb1, last_emb2, W1, b1, W2, b2,
           gru1_Wi, gru1_Wh, gru1_bi, gru1_bh,
           gru2_Wi, gru2_Wh, gru2_bi, gru2_bh,
           num_current_edges=_E, num_previous_edges=300000):
    pad = _EP - _E
    nrow = _EP // _CW
    pad_src = jnp.arange(pad, dtype=jnp.int32) % _N
    pad_dst = _N + (jnp.arange(pad, dtype=jnp.int32) % (_NP - _N))
    srcr = jnp.concatenate(
        [src.astype(jnp.int32), pad_src]
    ).reshape(nrow, _CW)
    dstr = jnp.concatenate(
        [dst.astype(jnp.int32), pad_dst]
    ).reshape(nrow, _CW)
    zeros_nh = jnp.zeros((_NP, _H), _f32)
    ones_h = jnp.ones((_CW, _H), _f32)

    degp = _sc_degree(dstr, zeros_nh, ones_h)
    y1, d8 = _tc_prescale(node_feat, W1, degp)
    aggp1 = _sc_aggregate(y1, srcr, dstr, zeros_nh)
    e1, y2 = _tc_layer1(aggp1, y1, d8, b1.reshape(1, _H), last_emb1,
                        gru1_Wi, gru1_Wh, gru1_bi.reshape(1, 3 * _H),
                        gru1_bh.reshape(1, 3 * _H), W2)
    aggp2 = _sc_aggregate(y2, srcr, dstr, zeros_nh)
    e2 = _tc_layer2(aggp2, y2, d8, b2.reshape(1, _H), last_emb2,
                    gru2_Wi, gru2_Wh, gru2_bi.reshape(1, 3 * _H),
                    gru2_bh.reshape(1, 3 * _H))
    return (e1, e2)
